# baseline (speedup 1.0000x reference)
import functools

import jax
import jax.numpy as jnp
from jax import lax
from jax.experimental import pallas as pl
from jax.experimental.pallas import tpu as pltpu
from jax.experimental.pallas import tpu_sc as plsc

F = 768
H = 12
DH = 64
K = 16
S = 2048
B = 2

BM = 512
BA = 512

NC = 2
NS = 16
NW = NC * NS
QPW = B * S // NW

BAND = 1.0e6
INITK = 3.0e38
PADK = 3.1e38
DONEK = 3.2e38
BIGI = 1.0e9


def _proj_kernel(x_ref, wq_ref, wk_ref, wv_ref, q_ref, k_ref, v_ref, ks_ref):
    i = pl.program_id(0)
    x = x_ref[...]
    q_ref[...] = jnp.dot(x, wq_ref[...], preferred_element_type=jnp.float32)
    kk = jnp.dot(x, wk_ref[...], preferred_element_type=jnp.float32)
    k_ref[...] = kk
    v_ref[...] = jnp.dot(x, wv_ref[...], preferred_element_type=jnp.float32)

    @pl.when(i % (S // BM) == 0)
    def _():
        ks_ref[...] = jnp.zeros_like(ks_ref)

    ks_ref[...] += jnp.sum(kk, axis=0, keepdims=True) * (1.0 / S)


_TAKE_DNUMS = lax.GatherDimensionNumbers(
    offset_dims=(), collapsed_slice_dims=(0,), start_index_map=(0,))


def _take(x, idx):
    return lax.gather(x, idx[:, None], _TAKE_DNUMS, slice_sizes=(1,),
                      mode=lax.GatherScatterMode.PROMISE_IN_BOUNDS)


def _bmax(x, iota):
    for s in (1, 2, 4, 8):
        x = jnp.maximum(x, _take(x, iota ^ s))
    return x


def _bmin(x, iota):
    for s in (1, 2, 4, 8):
        x = jnp.minimum(x, _take(x, iota ^ s))
    return x


def _sc_knn_kernel(cx_hbm, cy_hbm, cz_hbm, lens_hbm, out_hbm,
                   cx_v, cy_v, cz_v, out_v, lens_v):
    wid = lax.axis_index("s") * NC + lax.axis_index("c")
    b = wid // NS
    iota = lax.iota(jnp.int32, 16)

    pltpu.sync_copy(cx_hbm.at[b], cx_v)
    pltpu.sync_copy(cy_hbm.at[b], cy_v)
    pltpu.sync_copy(cz_hbm.at[b], cz_v)
    pltpu.sync_copy(lens_hbm, lens_v)

    nvi = _bmax(jnp.where(iota == jnp.full((16,), b, jnp.int32),
                          lens_v[...], 0), iota)
    nv = nvi.astype(jnp.float32)
    nvi_max = jnp.maximum(nvi, 33)

    def insert_body(c):
        m, key, jf, bk0, bk1, bi0, bi1, thr = c
        fi = plsc.all_reduce_ffs(m)
        civ = _take(jf, fi)
        ckv = _take(key, fi)
        m = m & (iota != fi)
        mkv = _bmax(jnp.maximum(bk0, bk1), iota)
        t0 = jnp.where(bk0 == mkv, bi0, -1.0)
        t1 = jnp.where(bk1 == mkv, bi1, -1.0)
        miv = _bmax(jnp.maximum(t0, t1), iota)
        e0 = (bk0 == mkv) & (bi0 == miv)
        e1 = (bk1 == mkv) & (bi1 == miv)
        bk0 = jnp.where(e0, ckv, bk0)
        bi0 = jnp.where(e0, civ, bi0)
        bk1 = jnp.where(e1, ckv, bk1)
        bi1 = jnp.where(e1, civ, bi1)
        thr = _bmax(jnp.maximum(bk0, bk1), iota)
        m = m & (key < thr)
        return m, key, jf, bk0, bk1, bi0, bi1, thr

    def valid_q(qq, q_local):
        qf = jnp.full((16,), q_local, jnp.int32)
        qx = plsc.load_gather(cx_v, [qf])
        qy = plsc.load_gather(cy_v, [qf])
        qz = plsc.load_gather(cz_v, [qf])
        iota_f = iota.astype(jnp.float32)

        def scan_cond(c):
            t = c[0]
            return jnp.any(jnp.full((16,), t * 16, jnp.int32) < nvi_max)

        def scan_body(c):
            t, bk0, bk1, bi0, bi1, thr = c
            off = pl.multiple_of(t * 16, 16)
            dx = cx_v[pl.ds(off, 16)] - qx
            dy = cy_v[pl.ds(off, 16)] - qy
            dz = cz_v[pl.ds(off, 16)] - qz
            d2 = dx * dx + dy * dy + dz * dz
            jf = (iota + t * 16).astype(jnp.float32)
            key = jnp.where(jf < nv, d2, BAND + jf)
            m = key < thr

            def do_insert(m, key, jf, bk0, bk1, bi0, bi1, thr):
                c2 = lax.while_loop(
                    lambda c: jnp.any(c[0]), insert_body,
                    (m, key, jf, bk0, bk1, bi0, bi1, thr))
                return c2[3:]

            bk0, bk1, bi0, bi1, thr = lax.cond(
                jnp.any(m), do_insert,
                lambda m, key, jf, bk0, bk1, bi0, bi1, thr:
                (bk0, bk1, bi0, bi1, thr),
                m, key, jf, bk0, bk1, bi0, bi1, thr)
            return t + 1, bk0, bk1, bi0, bi1, thr

        bk0 = jnp.full((16,), INITK, jnp.float32)
        bi0 = iota_f
        one = iota < 1
        bk1 = jnp.where(one, INITK, -1.0)
        bi1 = jnp.where(one, 16.0, -1.0)
        thr = jnp.full((16,), INITK, jnp.float32)
        _, bk0, bk1, bi0, bi1, thr = lax.while_loop(
            scan_cond, scan_body, (0, bk0, bk1, bi0, bi1, thr))

        bk1 = jnp.where(one, bk1, PADK)
        bi1 = jnp.where(one, bi1, BIGI)

        def ext_body(r, c):
            bk0, bk1, bi0, bi1, ov = c
            mnv = _bmin(jnp.minimum(bk0, bk1), iota)
            s0 = jnp.where(bk0 == mnv, bi0, BIGI)
            s1 = jnp.where(bk1 == mnv, bi1, BIGI)
            miv = _bmin(jnp.minimum(s0, s1), iota)
            rm0 = (bk0 == mnv) & (bi0 == miv)
            rm1 = (bk1 == mnv) & (bi1 == miv)
            bk0 = jnp.where(rm0, DONEK, bk0)
            bk1 = jnp.where(rm1, DONEK, bk1)
            ov = jnp.where(iota == jnp.full((16,), r - 1, jnp.int32), miv, ov)
            return bk0, bk1, bi0, bi1, ov

        ov = jnp.zeros((16,), jnp.float32)
        _, _, _, _, ov = lax.fori_loop(
            0, 17, ext_body, (bk0, bk1, bi0, bi1, ov))
        out_v[pl.ds(pl.multiple_of(qq * 16, 16), 16)] = ov.astype(jnp.int32)
        return 0

    def invalid_q(qq, q_local):
        out_v[pl.ds(pl.multiple_of(qq * 16, 16), 16)] = iota + 1
        return 0

    def q_body(qq, carry):
        q_local = wid * QPW + qq - b * S
        valid = jnp.any(jnp.full((16,), q_local, jnp.int32) < nvi)
        return lax.cond(valid, valid_q, invalid_q, qq, q_local)

    lax.fori_loop(0, QPW, q_body, 0)
    pltpu.sync_copy(out_v, out_hbm.at[pl.ds(wid * (QPW * K), QPW * K)])


def _attn_kernel(lens_ref, q_ref, k_ref, v_ref, nb_ref, out_ref):
    b = pl.program_id(0)
    n = lens_ref[b]
    jidx = lax.broadcasted_iota(jnp.int32, (BA, S), 1)

    mask = jnp.zeros((BA, S), jnp.bool_)
    for t in range(K):
        mask = jnp.logical_or(mask, jidx == nb_ref[:, t:t + 1])
    mask = jnp.logical_and(mask, jidx < n)

    scale = DH ** -0.5
    neg_inf = jnp.float32(-jnp.inf)
    for h in range(H):
        qh = q_ref[:, h * DH:(h + 1) * DH]
        kh = k_ref[:, h * DH:(h + 1) * DH]
        vh = v_ref[:, h * DH:(h + 1) * DH]
        logits = lax.dot_general(qh, kh, (((1,), (1,)), ((), ())),
                                 preferred_element_type=jnp.float32) * scale
        logits = jnp.where(mask, logits, neg_inf)
        rmax = jnp.max(logits, axis=1, keepdims=True)
        rmax = jnp.where(rmax > neg_inf, rmax, 0.0)
        p = jnp.where(mask, jnp.exp(logits - rmax), 0.0)
        denom = jnp.sum(p, axis=1, keepdims=True)
        p = p / jnp.where(denom > 0, denom, 1.0)
        out_ref[:, h * DH:(h + 1) * DH] = jnp.dot(
            p, vh, preferred_element_type=jnp.float32)


def kernel(x, coords, lens, Wq, Wk, Wv):
    x2d = x.reshape(B * S, F)
    lens = lens.astype(jnp.int32)

    q2d, k2d, v2d, ksum = pl.pallas_call(
        _proj_kernel,
        grid=(B * S // BM,),
        in_specs=[
            pl.BlockSpec((BM, F), lambda i: (i, 0)),
            pl.BlockSpec((F, F), lambda i: (0, 0)),
            pl.BlockSpec((F, F), lambda i: (0, 0)),
            pl.BlockSpec((F, F), lambda i: (0, 0)),
        ],
        out_specs=[
            pl.BlockSpec((BM, F), lambda i: (i, 0)),
            pl.BlockSpec((BM, F), lambda i: (i, 0)),
            pl.BlockSpec((BM, F), lambda i: (i, 0)),
            pl.BlockSpec((None, 1, F), lambda i: (i // (S // BM), 0, 0)),
        ],
        out_shape=[
            jax.ShapeDtypeStruct((B * S, F), jnp.float32),
            jax.ShapeDtypeStruct((B * S, F), jnp.float32),
            jax.ShapeDtypeStruct((B * S, F), jnp.float32),
            jax.ShapeDtypeStruct((B, 1, F), jnp.float32),
        ],
    )(x2d, Wq.T, Wk.T, Wv.T)

    cx = coords[:, :, 0]
    cy = coords[:, :, 1]
    cz = coords[:, :, 2]
    lens_pad = jnp.zeros((16,), jnp.int32).at[:B].set(lens)

    knn = pl.kernel(
        _sc_knn_kernel,
        mesh=plsc.VectorSubcoreMesh(core_axis_name="c", subcore_axis_name="s"),
        compiler_params=pltpu.CompilerParams(needs_layout_passes=False),
        out_type=jax.ShapeDtypeStruct((B * S * K,), jnp.int32),
        scratch_types=[
            pltpu.VMEM((S,), jnp.float32),
            pltpu.VMEM((S,), jnp.float32),
            pltpu.VMEM((S,), jnp.float32),
            pltpu.VMEM((QPW * K,), jnp.int32),
            pltpu.VMEM((16,), jnp.int32),
        ],
    )
    neigh = knn(cx, cy, cz, lens_pad).reshape(B, S, K)

    q3 = q2d.reshape(B, S, F)
    k3 = k2d.reshape(B, S, F)
    v3 = v2d.reshape(B, S, F)

    out = pl.pallas_call(
        _attn_kernel,
        grid=(B, S // BA),
        in_specs=[
            pl.BlockSpec(memory_space=pltpu.SMEM),
            pl.BlockSpec((None, BA, F), lambda b, i: (b, i, 0)),
            pl.BlockSpec((None, S, F), lambda b, i: (b, 0, 0)),
            pl.BlockSpec((None, S, F), lambda b, i: (b, 0, 0)),
            pl.BlockSpec((None, BA, K), lambda b, i: (b, i, 0)),
        ],
        out_specs=pl.BlockSpec((None, BA, F), lambda b, i: (b, i, 0)),
        out_shape=jax.ShapeDtypeStruct((B, S, F), jnp.float32),
    )(lens, q3, k3, v3, neigh)

    metric = ksum.reshape(B, H, DH)
    return (out, metric)
